# Initial kernel scaffold; baseline (speedup 1.0000x reference)
import jax
import jax.numpy as jnp
from jax.experimental import pallas as pl


def kernel(pred_masks, pred_keypoints):
    raise NotImplementedError("write your pallas kernel here")


# speedup vs baseline: 6.1959x; 6.1959x over previous
import jax
import jax.numpy as jnp
from jax import lax
from jax.experimental import pallas as pl
from jax.experimental.pallas import tpu as pltpu

_J2P = [[0, 1, 2, 3, 4], [5, 6, 11, 12], [5, 7, 9], [6, 8, 10],
        [11, 13, 15], [12, 14, 16], [15, 17, 18, 19], [16, 20, 21, 22]]
_P = 8
_H = 384
_W = 384


def _body(mask_ref, kp_ref, out_ref, acc_ref):
    b = pl.program_id(0)
    nb = pl.num_programs(0)

    rows = lax.broadcasted_iota(jnp.int32, (_H, _W), 0).astype(jnp.float32)
    cols = lax.broadcasted_iota(jnp.int32, (_H, _W), 1).astype(jnp.float32)

    def sums(m, thresh):
        pos = (m > thresh).astype(jnp.float32)
        cnt = jnp.sum(pos)
        sx = jnp.sum(pos * rows)
        sy = jnp.sum(pos * cols)
        return cnt, sx, sy

    def center(cnt, sx, sy):
        cx = jnp.where(cnt > 0, sx / jnp.maximum(cnt, 1.0), 0.0)
        cy = jnp.where(cnt > 0, sy / jnp.maximum(cnt, 1.0), 0.0)
        cx = jnp.where(cx > 0, cx, 0.0)
        cy = jnp.where(cy > 0, cy, 0.0)
        return cx, cy

    num = jnp.float32(0.0)
    den = jnp.float32(0.0)
    for p in range(_P):
        mcx, mcy = center(*sums(mask_ref[0, p + 1], 0.5))
        s = kp_ref[0, _J2P[p][0]]
        for j in _J2P[p][1:]:
            s = s + kp_ref[0, j]
        kcx, kcy = center(*sums(s, 0.3))
        valid = jnp.where(
            (kcx == 0.0) | (kcy == 0.0) | (mcx == 0.0) | (mcy == 0.0),
            0.0, 1.0)
        num += valid * ((mcx - kcx) ** 2 + (mcy - kcy) ** 2)
        den += 2.0 * valid

    @pl.when(b == 0)
    def _():
        acc_ref[0] = num
        acc_ref[1] = den

    @pl.when(b != 0)
    def _():
        acc_ref[0] = acc_ref[0] + num
        acc_ref[1] = acc_ref[1] + den

    loss = jnp.float32(1e-5) * acc_ref[0] / jnp.maximum(acc_ref[1], 1.0)
    out_ref[...] = loss.reshape(1, 1)


@jax.jit
def kernel(pred_masks, pred_keypoints):
    out = pl.pallas_call(
        _body,
        grid=(16,),
        in_specs=[
            pl.BlockSpec((1, 9, _H, _W), lambda b: (b, 0, 0, 0)),
            pl.BlockSpec((1, 23, _H, _W), lambda b: (b, 0, 0, 0)),
        ],
        out_specs=pl.BlockSpec((1, 1), lambda b: (0, 0)),
        out_shape=jax.ShapeDtypeStruct((1, 1), jnp.float32),
        scratch_shapes=[pltpu.SMEM((2,), jnp.float32)],
        compiler_params=pltpu.CompilerParams(
            vmem_limit_bytes=100 * 1024 * 1024),
    )(pred_masks, pred_keypoints)
    return out[0, 0]
